# Initial kernel scaffold; baseline (speedup 1.0000x reference)
import jax
import jax.numpy as jnp
from jax.experimental import pallas as pl


def kernel(x, W, att_src, att_dst, bias):
    raise NotImplementedError("write your pallas kernel here")


# speedup vs baseline: 14267.7117x; 14267.7117x over previous
import jax
import jax.numpy as jnp
from jax.experimental import pallas as pl

N = 2048
BJ = 256
NEG_SLOPE = 0.2


def _gat_block(xc_ref, xr_ref, w_ref, as_ref, ad_ref, b_ref, out_ref):
    w = w_ref[0, :]
    s = jnp.sum(w * as_ref[0, :])
    t = jnp.sum(w * ad_ref[0, :])
    wbar = jnp.mean(w)
    bbar = jnp.mean(b_ref[0, :])

    xc = xc_ref[:, :]
    u = s * xc
    c = t * xr_ref[:, :]
    e = u + c
    e = jnp.where(e > 0, e, NEG_SLOPE * e)
    m = jnp.max(e, axis=0, keepdims=True)
    p = jnp.exp(e - m)
    z = jnp.sum(p, axis=0, keepdims=True)
    num = jnp.sum(p * xc, axis=0, keepdims=True)
    out_ref[:, :] = wbar * num / (z + 1e-16) + bbar


def kernel(x, W, att_src, att_dst, bias):
    a, b, n, d = x.shape
    xf = x.reshape(n, 1)
    xr = x.reshape(1, n)
    w2 = W.reshape(1, -1)
    as2 = att_src.reshape(1, -1)
    ad2 = att_dst.reshape(1, -1)
    b2 = bias.reshape(1, -1)

    out = pl.pallas_call(
        _gat_block,
        grid=(n // BJ,),
        in_specs=[
            pl.BlockSpec((n, 1), lambda j: (0, 0)),
            pl.BlockSpec((1, BJ), lambda j: (0, j)),
            pl.BlockSpec(w2.shape, lambda j: (0, 0)),
            pl.BlockSpec(as2.shape, lambda j: (0, 0)),
            pl.BlockSpec(ad2.shape, lambda j: (0, 0)),
            pl.BlockSpec(b2.shape, lambda j: (0, 0)),
        ],
        out_specs=pl.BlockSpec((1, BJ), lambda j: (0, j)),
        out_shape=jax.ShapeDtypeStruct((1, n), jnp.float32),
    )(xf, xr, w2, as2, ad2, b2)

    return out.reshape(n, a, b, d).transpose(1, 2, 0, 3)
